# Initial kernel scaffold; baseline (speedup 1.0000x reference)
import jax
import jax.numpy as jnp
from jax.experimental import pallas as pl


def kernel(x, edge_index, batch, ptr, Wl1, Wr1, b1, Wl2, Wr2, b2, W_lin1, b_lin1, gamma, beta, W_lin2, b_lin2):
    raise NotImplementedError("write your pallas kernel here")


# speedup vs baseline: 7.0689x; 7.0689x over previous
import functools
import math

import jax
import jax.numpy as jnp
from jax import lax
from jax.experimental import pallas as pl
from jax.experimental.pallas import tpu as pltpu
from jax.experimental.pallas import tpu_sc as plsc

N = 10000
E = 320000
D = 128
H = 128
G = 64
HD8 = H // 8
C = 2

NC = 2
NS = 16
NW = NC * NS
CH = 80
EPW = E // NW
NCH = EPW // CH
RPT = N // NS

_LEAKY = 0.2
_BN_SCALE = 1.0 / math.sqrt(1.0 + 1e-5)


def _leaky(v):
    return jnp.where(v > 0, v, _LEAKY * v)


@functools.lru_cache(maxsize=None)
def _make_sc_scatter(W):
    mesh = plsc.VectorSubcoreMesh(
        core_axis_name="c", subcore_axis_name="s", num_cores=NC, num_subcores=NS
    )

    @functools.partial(
        pl.kernel,
        out_type=jax.ShapeDtypeStruct((NC, N, W), jnp.float32),
        mesh=mesh,
        scratch_types=[
            pltpu.VMEM((NCH, CH), jnp.int32),
            pltpu.VMEM((NCH, CH), jnp.int32),
            pltpu.VMEM((CH, W), jnp.float32),
            pltpu.VMEM_SHARED((N, W), jnp.float32),
            pltpu.SemaphoreType.DMA,
        ],
        compiler_params=pltpu.CompilerParams(use_tc_tiling_on_sc=False),
    )
    def sc_pass(table_hbm, src_hbm, dst_hbm, zeros_hbm, out_hbm,
                src_v, dst_v, buf, acc_sh, sem):
        c = lax.axis_index("c")
        s = lax.axis_index("s")
        w = c * NS + s

        pltpu.sync_copy(zeros_hbm.at[pl.ds(s * RPT, RPT)],
                        acc_sh.at[pl.ds(s * RPT, RPT)])
        pltpu.sync_copy(src_hbm.at[pl.ds(w * NCH, NCH)], src_v)
        pltpu.sync_copy(dst_hbm.at[pl.ds(w * NCH, NCH)], dst_v)
        plsc.subcore_barrier()

        def body(j, carry):
            pltpu.async_copy(table_hbm.at[src_v.at[j]], buf, sem).wait()
            pltpu.sync_copy(buf, acc_sh.at[dst_v.at[j]], add=True)
            return carry

        lax.fori_loop(0, NCH, body, 0)
        plsc.subcore_barrier()

        pltpu.sync_copy(acc_sh.at[pl.ds(s * RPT, RPT)],
                        out_hbm.at[c, pl.ds(s * RPT, RPT)])

    return sc_pass


_B1 = 1000
_G1 = N // _B1


def _tc1_body(x_ref, p0_ref, p1_ref, wl_ref, wr_ref, b_ref, h_ref, rdeg_ref):
    agg = p0_ref[...] + p1_ref[...]
    deg = agg[:, D:D + 1]
    rdeg = 1.0 / jnp.maximum(deg, 1.0)
    mean = agg[:, :D] * rdeg
    acc = lax.dot_general(mean, wl_ref[...], (((1,), (1,)), ((), ())),
                          preferred_element_type=jnp.float32)
    acc += lax.dot_general(x_ref[...], wr_ref[...], (((1,), (1,)), ((), ())),
                           preferred_element_type=jnp.float32)
    h_ref[...] = _leaky(acc + b_ref[...])
    rdeg_ref[...] = rdeg


def _tc1(x, p0, p1, wl, wr, b):
    return pl.pallas_call(
        _tc1_body,
        grid=(_G1,),
        in_specs=[
            pl.BlockSpec((_B1, D), lambda i: (i, 0)),
            pl.BlockSpec((_B1, 144), lambda i: (i, 0)),
            pl.BlockSpec((_B1, 144), lambda i: (i, 0)),
            pl.BlockSpec((H, D), lambda i: (0, 0)),
            pl.BlockSpec((H, D), lambda i: (0, 0)),
            pl.BlockSpec((1, H), lambda i: (0, 0)),
        ],
        out_specs=[
            pl.BlockSpec((_B1, H), lambda i: (i, 0)),
            pl.BlockSpec((_B1, 1), lambda i: (i, 0)),
        ],
        out_shape=[
            jax.ShapeDtypeStruct((N, H), jnp.float32),
            jax.ShapeDtypeStruct((N, 1), jnp.float32),
        ],
    )(x, p0, p1, wl, wr, b)


def _tc2_body(h_ref, p0_ref, p1_ref, rdeg_ref, batch_ref, large_ref,
              wl_ref, wr_ref, b_ref, pool_ref, cnt_ref):
    step = pl.program_id(0)
    hb = h_ref[...]
    mean2 = (p0_ref[...] + p1_ref[...]) * rdeg_ref[...]
    acc = lax.dot_general(mean2, wl_ref[...], (((1,), (1,)), ((), ())),
                          preferred_element_type=jnp.float32)
    acc += lax.dot_general(hb, wr_ref[...], (((1,), (1,)), ((), ())),
                           preferred_element_type=jnp.float32)
    h2 = _leaky(acc + b_ref[...])

    iota = lax.broadcasted_iota(jnp.int32, (_B1, G), 1)
    m = (batch_ref[...] == iota).astype(jnp.float32)
    is_large = lax.dot_general(m, large_ref[...], (((1,), (1,)), ((), ())),
                               preferred_element_type=jnp.float32)
    hf = is_large * h2 + (1.0 - is_large) * hb

    ps = lax.dot_general(m, hf, (((0,), (0,)), ((), ())),
                         preferred_element_type=jnp.float32)
    cs = lax.dot_general(m, jnp.ones_like(hf), (((0,), (0,)), ((), ())),
                         preferred_element_type=jnp.float32)

    @pl.when(step == 0)
    def _():
        pool_ref[...] = jnp.zeros_like(pool_ref)
        cnt_ref[...] = jnp.zeros_like(cnt_ref)

    pool_ref[...] += ps
    cnt_ref[...] += cs


def _tc2(h, p0, p1, rdeg, batch2d, large, wl, wr, b):
    return pl.pallas_call(
        _tc2_body,
        grid=(_G1,),
        in_specs=[
            pl.BlockSpec((_B1, H), lambda i: (i, 0)),
            pl.BlockSpec((_B1, H), lambda i: (i, 0)),
            pl.BlockSpec((_B1, H), lambda i: (i, 0)),
            pl.BlockSpec((_B1, 1), lambda i: (i, 0)),
            pl.BlockSpec((_B1, 1), lambda i: (i, 0)),
            pl.BlockSpec((1, G), lambda i: (0, 0)),
            pl.BlockSpec((H, H), lambda i: (0, 0)),
            pl.BlockSpec((H, H), lambda i: (0, 0)),
            pl.BlockSpec((1, H), lambda i: (0, 0)),
        ],
        out_specs=[
            pl.BlockSpec((G, H), lambda i: (0, 0)),
            pl.BlockSpec((G, H), lambda i: (0, 0)),
        ],
        out_shape=[
            jax.ShapeDtypeStruct((G, H), jnp.float32),
            jax.ShapeDtypeStruct((G, H), jnp.float32),
        ],
    )(h, p0, p1, rdeg, batch2d, large, wl, wr, b)


def _tc3_body(pool_ref, cnt_ref, w1_ref, b1_ref, g_ref, be_ref, w2_ref,
              b2_ref, out_ref):
    pooled = pool_ref[...] / jnp.maximum(cnt_ref[...], 1.0)
    z = lax.dot_general(pooled, w1_ref[...], (((1,), (1,)), ((), ())),
                        preferred_element_type=jnp.float32) + b1_ref[...]
    z = z * _BN_SCALE * g_ref[...] + be_ref[...]
    z = _leaky(z)
    out_ref[...] = lax.dot_general(z, w2_ref[...], (((1,), (1,)), ((), ())),
                                   preferred_element_type=jnp.float32) + b2_ref[...]


def _tc3(pool, cnt, w1, b1, g, be, w2, b2):
    return pl.pallas_call(
        _tc3_body,
        out_shape=jax.ShapeDtypeStruct((G, C), jnp.float32),
    )(pool, cnt, w1, b1, g, be, w2, b2)


def kernel(x, edge_index, batch, ptr, Wl1, Wr1, b1, Wl2, Wr2, b2,
           W_lin1, b_lin1, gamma, beta, W_lin2, b_lin2):
    src2d = edge_index[0].reshape(E // CH, CH)
    dst2d = edge_index[1].reshape(E // CH, CH)

    xpad = jnp.concatenate(
        [x, jnp.ones((N, 1), jnp.float32), jnp.zeros((N, 15), jnp.float32)],
        axis=1)
    z144 = jnp.zeros((N, 144), jnp.float32)
    z128 = jnp.zeros((N, H), jnp.float32)

    part1 = _make_sc_scatter(144)(xpad, src2d, dst2d, z144)
    h, rdeg = _tc1(x, part1[0], part1[1], Wl1, Wr1, b1.reshape(1, H))

    part2 = _make_sc_scatter(128)(h, src2d, dst2d, z128)

    large = (ptr[1:] - ptr[:-1] >= 40).astype(jnp.float32).reshape(1, G)
    pool, cnt = _tc2(h, part2[0], part2[1], rdeg, batch.reshape(N, 1),
                     large, Wl2, Wr2, b2.reshape(1, H))

    return _tc3(pool, cnt, W_lin1, b_lin1.reshape(1, HD8),
                gamma.reshape(1, HD8), beta.reshape(1, HD8),
                W_lin2, b_lin2.reshape(1, C))


# speedup vs baseline: 8.9228x; 1.2623x over previous
import functools
import math

import jax
import jax.numpy as jnp
from jax import lax
from jax.experimental import pallas as pl
from jax.experimental.pallas import tpu as pltpu
from jax.experimental.pallas import tpu_sc as plsc

N = 10000
E = 320000
D = 128
H = 128
G = 64
HD8 = H // 8
C = 2

NC = 2
NS = 16
NW = NC * NS
CH = 125
EPW = E // NW
NCH = EPW // CH
NBUF = 2
NGRP = NCH // NBUF
RPT = N // NS

_LEAKY = 0.2
_BN_SCALE = 1.0 / math.sqrt(1.0 + 1e-5)


def _leaky(v):
    return jnp.where(v > 0, v, _LEAKY * v)


@functools.lru_cache(maxsize=None)
def _make_sc_scatter(W):
    mesh = plsc.VectorSubcoreMesh(
        core_axis_name="c", subcore_axis_name="s", num_cores=NC, num_subcores=NS
    )

    @functools.partial(
        pl.kernel,
        out_type=jax.ShapeDtypeStruct((NC, N, W), jnp.float32),
        mesh=mesh,
        scratch_types=[
            pltpu.VMEM((2, NBUF, CH), jnp.int32),
            pltpu.VMEM((2, NBUF, CH), jnp.int32),
            pltpu.VMEM((NBUF, CH, W), jnp.float32),
            pltpu.VMEM_SHARED((N, W), jnp.float32),
            [pltpu.SemaphoreType.DMA] * 2,
            [pltpu.SemaphoreType.DMA] * NBUF,
            [pltpu.SemaphoreType.DMA] * NBUF,
        ],
        compiler_params=pltpu.CompilerParams(use_tc_tiling_on_sc=False),
    )
    def sc_pass(table_hbm, src_hbm, dst_hbm, zeros_hbm, out_hbm,
                sidx, didx, rbuf, acc_sh, isems, gsems, ssems):
        c = lax.axis_index("c")
        s = lax.axis_index("s")
        w = c * NS + s

        def idx_copy(g, ib):
            pltpu.async_copy(src_hbm.at[w, g], sidx.at[ib], isems[ib])
            pltpu.async_copy(dst_hbm.at[w, g], didx.at[ib], isems[ib])

        def idx_wait(g, ib):
            pltpu.make_async_copy(src_hbm.at[w, g], sidx.at[ib],
                                  isems[ib]).wait()
            pltpu.make_async_copy(dst_hbm.at[w, g], didx.at[ib],
                                  isems[ib]).wait()

        def gather(ib, b):
            pltpu.async_copy(table_hbm.at[sidx.at[ib, b]], rbuf.at[b],
                             gsems[b])

        def gather_wait(ib, b):
            pltpu.make_async_copy(table_hbm.at[sidx.at[ib, b]], rbuf.at[b],
                                  gsems[b]).wait()

        def scatter(ib, b):
            pltpu.async_copy(rbuf.at[b], acc_sh.at[didx.at[ib, b]], ssems[b],
                             add=True)

        def scatter_wait(ib, b):
            pltpu.make_async_copy(rbuf.at[b], acc_sh.at[didx.at[ib, b]],
                                  ssems[b]).wait()

        pltpu.sync_copy(zeros_hbm.at[pl.ds(s * RPT, RPT)],
                        acc_sh.at[pl.ds(s * RPT, RPT)])

        idx_copy(0, 0)
        idx_wait(0, 0)
        for b in range(NBUF):
            gather(0, b)
        idx_copy(1, 1)
        plsc.subcore_barrier()

        def pair_body(i, carry):
            for ib in range(2):
                g = 2 * i + ib

                @pl.when(g + 1 < NGRP)
                def _():
                    idx_wait(g + 1, 1 - ib)

                for b in range(NBUF):
                    gather_wait(ib, b)
                    scatter(ib, b)

                for b in range(NBUF):
                    scatter_wait(ib, b)

                    @pl.when(g + 1 < NGRP)
                    def _(b=b, ib=ib):
                        gather(1 - ib, b)

                @pl.when(g + 2 < NGRP)
                def _():
                    idx_copy(g + 2, ib)
            return carry

        lax.fori_loop(0, NGRP // 2, pair_body, 0)
        plsc.subcore_barrier()

        pltpu.sync_copy(acc_sh.at[pl.ds(s * RPT, RPT)],
                        out_hbm.at[c, pl.ds(s * RPT, RPT)])

    return sc_pass


_B1 = 1000
_G1 = N // _B1


def _tc1_body(x_ref, p0_ref, p1_ref, wl_ref, wr_ref, b_ref, h_ref, rdeg_ref):
    agg = p0_ref[...] + p1_ref[...]
    deg = agg[:, D:D + 1]
    rdeg = 1.0 / jnp.maximum(deg, 1.0)
    mean = agg[:, :D] * rdeg
    acc = lax.dot_general(mean, wl_ref[...], (((1,), (1,)), ((), ())),
                          preferred_element_type=jnp.float32)
    acc += lax.dot_general(x_ref[...], wr_ref[...], (((1,), (1,)), ((), ())),
                           preferred_element_type=jnp.float32)
    h_ref[...] = _leaky(acc + b_ref[...])
    rdeg_ref[...] = rdeg


def _tc1(x, p0, p1, wl, wr, b):
    return pl.pallas_call(
        _tc1_body,
        grid=(_G1,),
        in_specs=[
            pl.BlockSpec((_B1, D), lambda i: (i, 0)),
            pl.BlockSpec((_B1, 144), lambda i: (i, 0)),
            pl.BlockSpec((_B1, 144), lambda i: (i, 0)),
            pl.BlockSpec((H, D), lambda i: (0, 0)),
            pl.BlockSpec((H, D), lambda i: (0, 0)),
            pl.BlockSpec((1, H), lambda i: (0, 0)),
        ],
        out_specs=[
            pl.BlockSpec((_B1, H), lambda i: (i, 0)),
            pl.BlockSpec((_B1, 1), lambda i: (i, 0)),
        ],
        out_shape=[
            jax.ShapeDtypeStruct((N, H), jnp.float32),
            jax.ShapeDtypeStruct((N, 1), jnp.float32),
        ],
    )(x, p0, p1, wl, wr, b)


def _tc2_body(h_ref, p0_ref, p1_ref, rdeg_ref, batch_ref, large_ref,
              wl_ref, wr_ref, b_ref, w1_ref, b1_ref, g_ref, be_ref,
              w2_ref, b2_ref, out_ref, pool_ref, cnt_ref):
    step = pl.program_id(0)
    hb = h_ref[...]
    mean2 = (p0_ref[...] + p1_ref[...]) * rdeg_ref[...]
    acc = lax.dot_general(mean2, wl_ref[...], (((1,), (1,)), ((), ())),
                          preferred_element_type=jnp.float32)
    acc += lax.dot_general(hb, wr_ref[...], (((1,), (1,)), ((), ())),
                           preferred_element_type=jnp.float32)
    h2 = _leaky(acc + b_ref[...])

    iota = lax.broadcasted_iota(jnp.int32, (_B1, G), 1)
    m = (batch_ref[...] == iota).astype(jnp.float32)
    is_large = lax.dot_general(m, large_ref[...], (((1,), (1,)), ((), ())),
                               preferred_element_type=jnp.float32)
    hf = is_large * h2 + (1.0 - is_large) * hb

    ps = lax.dot_general(m, hf, (((0,), (0,)), ((), ())),
                         preferred_element_type=jnp.float32)
    cs = lax.dot_general(m, jnp.ones_like(hf), (((0,), (0,)), ((), ())),
                         preferred_element_type=jnp.float32)

    @pl.when(step == 0)
    def _():
        pool_ref[...] = jnp.zeros_like(pool_ref)
        cnt_ref[...] = jnp.zeros_like(cnt_ref)

    pool_ref[...] += ps
    cnt_ref[...] += cs

    @pl.when(step == _G1 - 1)
    def _():
        pooled = pool_ref[...] / jnp.maximum(cnt_ref[...], 1.0)
        z = lax.dot_general(pooled, w1_ref[...], (((1,), (1,)), ((), ())),
                            preferred_element_type=jnp.float32) + b1_ref[...]
        z = z * _BN_SCALE * g_ref[...] + be_ref[...]
        z = _leaky(z)
        out_ref[...] = lax.dot_general(z, w2_ref[...], (((1,), (1,)), ((), ())),
                                       preferred_element_type=jnp.float32) + b2_ref[...]


def _tc2(h, p0, p1, rdeg, batch2d, large, wl, wr, b, w1, b1, g, be, w2, b2):
    return pl.pallas_call(
        _tc2_body,
        grid=(_G1,),
        in_specs=[
            pl.BlockSpec((_B1, H), lambda i: (i, 0)),
            pl.BlockSpec((_B1, H), lambda i: (i, 0)),
            pl.BlockSpec((_B1, H), lambda i: (i, 0)),
            pl.BlockSpec((_B1, 1), lambda i: (i, 0)),
            pl.BlockSpec((_B1, 1), lambda i: (i, 0)),
            pl.BlockSpec((1, G), lambda i: (0, 0)),
            pl.BlockSpec((H, H), lambda i: (0, 0)),
            pl.BlockSpec((H, H), lambda i: (0, 0)),
            pl.BlockSpec((1, H), lambda i: (0, 0)),
            pl.BlockSpec((HD8, H), lambda i: (0, 0)),
            pl.BlockSpec((1, HD8), lambda i: (0, 0)),
            pl.BlockSpec((1, HD8), lambda i: (0, 0)),
            pl.BlockSpec((1, HD8), lambda i: (0, 0)),
            pl.BlockSpec((C, HD8), lambda i: (0, 0)),
            pl.BlockSpec((1, C), lambda i: (0, 0)),
        ],
        out_specs=pl.BlockSpec((G, C), lambda i: (0, 0)),
        out_shape=jax.ShapeDtypeStruct((G, C), jnp.float32),
        scratch_shapes=[
            pltpu.VMEM((G, H), jnp.float32),
            pltpu.VMEM((G, H), jnp.float32),
        ],
    )(h, p0, p1, rdeg, batch2d, large, wl, wr, b, w1, b1, g, be, w2, b2)


def kernel(x, edge_index, batch, ptr, Wl1, Wr1, b1, Wl2, Wr2, b2,
           W_lin1, b_lin1, gamma, beta, W_lin2, b_lin2):
    src2d = edge_index[0].reshape(NW, NGRP, NBUF, CH)
    dst2d = edge_index[1].reshape(NW, NGRP, NBUF, CH)

    xpad = jnp.concatenate(
        [x, jnp.ones((N, 1), jnp.float32), jnp.zeros((N, 15), jnp.float32)],
        axis=1)
    z144 = jnp.zeros((N, 144), jnp.float32)
    z128 = jnp.zeros((N, H), jnp.float32)

    part1 = _make_sc_scatter(144)(xpad, src2d, dst2d, z144)
    h, rdeg = _tc1(x, part1[0], part1[1], Wl1, Wr1, b1.reshape(1, H))

    part2 = _make_sc_scatter(128)(h, src2d, dst2d, z128)

    large = (ptr[1:] - ptr[:-1] >= 40).astype(jnp.float32).reshape(1, G)
    return _tc2(h, part2[0], part2[1], rdeg, batch.reshape(N, 1),
                large, Wl2, Wr2, b2.reshape(1, H),
                W_lin1, b_lin1.reshape(1, HD8),
                gamma.reshape(1, HD8), beta.reshape(1, HD8),
                W_lin2, b_lin2.reshape(1, C))


# speedup vs baseline: 9.0337x; 1.0124x over previous
import functools
import math

import jax
import jax.numpy as jnp
from jax import lax
from jax.experimental import pallas as pl
from jax.experimental.pallas import tpu as pltpu
from jax.experimental.pallas import tpu_sc as plsc

N = 10000
E = 320000
D = 128
H = 128
G = 64
HD8 = H // 8
C = 2

NC = 2
NS = 16
NW = NC * NS
CH = 100
EPW = E // NW
NCH = EPW // CH
RPT = N // NS

_LEAKY = 0.2
_BN_SCALE = 1.0 / math.sqrt(1.0 + 1e-5)


def _leaky(v):
    return jnp.where(v > 0, v, _LEAKY * v)


@functools.lru_cache(maxsize=None)
def _make_sc_scatter(W):
    mesh = plsc.VectorSubcoreMesh(
        core_axis_name="c", subcore_axis_name="s", num_cores=NC, num_subcores=NS
    )

    @functools.partial(
        pl.kernel,
        out_type=jax.ShapeDtypeStruct((NC, N, W), jnp.float32),
        mesh=mesh,
        scratch_types=[
            pltpu.VMEM((2, CH), jnp.int32),
            pltpu.VMEM((NCH, CH), jnp.int32),
            pltpu.VMEM((2, CH, W), jnp.float32),
            pltpu.VMEM_SHARED((N, W), jnp.float32),
            [pltpu.SemaphoreType.DMA] * 2,
            [pltpu.SemaphoreType.DMA] * 2,
            [pltpu.SemaphoreType.DMA] * 2,
        ],
        compiler_params=pltpu.CompilerParams(use_tc_tiling_on_sc=False),
    )
    def sc_pass(table_hbm, src_hbm, dst_hbm, zeros_hbm, out_hbm,
                sidx, didx, rbuf, acc_sh, isems, gsems, ssems):
        c = lax.axis_index("c")
        s = lax.axis_index("s")
        w = c * NS + s

        def idx_copy(j, p):
            pltpu.async_copy(src_hbm.at[w, j], sidx.at[p], isems[p])

        def idx_wait(j, p):
            pltpu.make_async_copy(src_hbm.at[w, j], sidx.at[p],
                                  isems[p]).wait()

        def gather(p):
            pltpu.async_copy(table_hbm.at[sidx.at[p]], rbuf.at[p], gsems[p])

        def gather_wait(p):
            pltpu.make_async_copy(table_hbm.at[sidx.at[p]], rbuf.at[p],
                                  gsems[p]).wait()

        def scatter(j, p):
            pltpu.async_copy(rbuf.at[p], acc_sh.at[didx.at[j]], ssems[p],
                             add=True)

        def scatter_wait(j, p):
            pltpu.make_async_copy(rbuf.at[p], acc_sh.at[didx.at[j]],
                                  ssems[p]).wait()

        pltpu.sync_copy(zeros_hbm.at[pl.ds(s * RPT, RPT)],
                        acc_sh.at[pl.ds(s * RPT, RPT)])
        pltpu.sync_copy(dst_hbm.at[w], didx)
        idx_copy(0, 0)
        idx_copy(1, 1)
        idx_wait(0, 0)
        gather(0)
        plsc.subcore_barrier()

        def pair_body(i, carry):
            for p in range(2):
                j = 2 * i + p
                gather_wait(p)
                scatter(j, p)

                @pl.when(j + 2 < NCH)
                def _():
                    idx_copy(j + 2, p)

                @pl.when(j >= 1)
                def _():
                    scatter_wait(j - 1, 1 - p)

                @pl.when(j + 1 < NCH)
                def _():
                    idx_wait(j + 1, 1 - p)
                    gather(1 - p)
            return carry

        lax.fori_loop(0, NCH // 2, pair_body, 0)
        scatter_wait(NCH - 1, 1)
        plsc.subcore_barrier()

        pltpu.sync_copy(acc_sh.at[pl.ds(s * RPT, RPT)],
                        out_hbm.at[c, pl.ds(s * RPT, RPT)])

    return sc_pass


_B1 = 1000
_G1 = N // _B1


def _tc1_body(x_ref, p0_ref, p1_ref, wl_ref, wr_ref, b_ref, h_ref, rdeg_ref):
    agg = p0_ref[...] + p1_ref[...]
    deg = agg[:, D:D + 1]
    rdeg = 1.0 / jnp.maximum(deg, 1.0)
    mean = agg[:, :D] * rdeg
    acc = lax.dot_general(mean, wl_ref[...], (((1,), (1,)), ((), ())),
                          preferred_element_type=jnp.float32)
    acc += lax.dot_general(x_ref[...], wr_ref[...], (((1,), (1,)), ((), ())),
                           preferred_element_type=jnp.float32)
    h_ref[...] = _leaky(acc + b_ref[...])
    rdeg_ref[...] = rdeg


def _tc1(x, p0, p1, wl, wr, b):
    return pl.pallas_call(
        _tc1_body,
        grid=(_G1,),
        in_specs=[
            pl.BlockSpec((_B1, D), lambda i: (i, 0)),
            pl.BlockSpec((_B1, 144), lambda i: (i, 0)),
            pl.BlockSpec((_B1, 144), lambda i: (i, 0)),
            pl.BlockSpec((H, D), lambda i: (0, 0)),
            pl.BlockSpec((H, D), lambda i: (0, 0)),
            pl.BlockSpec((1, H), lambda i: (0, 0)),
        ],
        out_specs=[
            pl.BlockSpec((_B1, H), lambda i: (i, 0)),
            pl.BlockSpec((_B1, 1), lambda i: (i, 0)),
        ],
        out_shape=[
            jax.ShapeDtypeStruct((N, H), jnp.float32),
            jax.ShapeDtypeStruct((N, 1), jnp.float32),
        ],
    )(x, p0, p1, wl, wr, b)


def _tc2_body(h_ref, p0_ref, p1_ref, rdeg_ref, batch_ref, large_ref,
              wl_ref, wr_ref, b_ref, w1_ref, b1_ref, g_ref, be_ref,
              w2_ref, b2_ref, out_ref, pool_ref, cnt_ref):
    step = pl.program_id(0)
    hb = h_ref[...]
    mean2 = (p0_ref[...] + p1_ref[...]) * rdeg_ref[...]
    acc = lax.dot_general(mean2, wl_ref[...], (((1,), (1,)), ((), ())),
                          preferred_element_type=jnp.float32)
    acc += lax.dot_general(hb, wr_ref[...], (((1,), (1,)), ((), ())),
                           preferred_element_type=jnp.float32)
    h2 = _leaky(acc + b_ref[...])

    iota = lax.broadcasted_iota(jnp.int32, (_B1, G), 1)
    m = (batch_ref[...] == iota).astype(jnp.float32)
    is_large = lax.dot_general(m, large_ref[...], (((1,), (1,)), ((), ())),
                               preferred_element_type=jnp.float32)
    hf = is_large * h2 + (1.0 - is_large) * hb

    ps = lax.dot_general(m, hf, (((0,), (0,)), ((), ())),
                         preferred_element_type=jnp.float32)
    cs = lax.dot_general(m, jnp.ones_like(hf), (((0,), (0,)), ((), ())),
                         preferred_element_type=jnp.float32)

    @pl.when(step == 0)
    def _():
        pool_ref[...] = jnp.zeros_like(pool_ref)
        cnt_ref[...] = jnp.zeros_like(cnt_ref)

    pool_ref[...] += ps
    cnt_ref[...] += cs

    @pl.when(step == _G1 - 1)
    def _():
        pooled = pool_ref[...] / jnp.maximum(cnt_ref[...], 1.0)
        z = lax.dot_general(pooled, w1_ref[...], (((1,), (1,)), ((), ())),
                            preferred_element_type=jnp.float32) + b1_ref[...]
        z = z * _BN_SCALE * g_ref[...] + be_ref[...]
        z = _leaky(z)
        out_ref[...] = lax.dot_general(z, w2_ref[...], (((1,), (1,)), ((), ())),
                                       preferred_element_type=jnp.float32) + b2_ref[...]


def _tc2(h, p0, p1, rdeg, batch2d, large, wl, wr, b, w1, b1, g, be, w2, b2):
    return pl.pallas_call(
        _tc2_body,
        grid=(_G1,),
        in_specs=[
            pl.BlockSpec((_B1, H), lambda i: (i, 0)),
            pl.BlockSpec((_B1, H), lambda i: (i, 0)),
            pl.BlockSpec((_B1, H), lambda i: (i, 0)),
            pl.BlockSpec((_B1, 1), lambda i: (i, 0)),
            pl.BlockSpec((_B1, 1), lambda i: (i, 0)),
            pl.BlockSpec((1, G), lambda i: (0, 0)),
            pl.BlockSpec((H, H), lambda i: (0, 0)),
            pl.BlockSpec((H, H), lambda i: (0, 0)),
            pl.BlockSpec((1, H), lambda i: (0, 0)),
            pl.BlockSpec((HD8, H), lambda i: (0, 0)),
            pl.BlockSpec((1, HD8), lambda i: (0, 0)),
            pl.BlockSpec((1, HD8), lambda i: (0, 0)),
            pl.BlockSpec((1, HD8), lambda i: (0, 0)),
            pl.BlockSpec((C, HD8), lambda i: (0, 0)),
            pl.BlockSpec((1, C), lambda i: (0, 0)),
        ],
        out_specs=pl.BlockSpec((G, C), lambda i: (0, 0)),
        out_shape=jax.ShapeDtypeStruct((G, C), jnp.float32),
        scratch_shapes=[
            pltpu.VMEM((G, H), jnp.float32),
            pltpu.VMEM((G, H), jnp.float32),
        ],
    )(h, p0, p1, rdeg, batch2d, large, wl, wr, b, w1, b1, g, be, w2, b2)


def kernel(x, edge_index, batch, ptr, Wl1, Wr1, b1, Wl2, Wr2, b2,
           W_lin1, b_lin1, gamma, beta, W_lin2, b_lin2):
    src2d = edge_index[0].reshape(NW, NCH, CH)
    dst2d = edge_index[1].reshape(NW, NCH, CH)

    xpad = jnp.concatenate(
        [x, jnp.ones((N, 1), jnp.float32), jnp.zeros((N, 15), jnp.float32)],
        axis=1)
    z144 = jnp.zeros((N, 144), jnp.float32)
    z128 = jnp.zeros((N, H), jnp.float32)

    part1 = _make_sc_scatter(144)(xpad, src2d, dst2d, z144)
    h, rdeg = _tc1(x, part1[0], part1[1], Wl1, Wr1, b1.reshape(1, H))

    part2 = _make_sc_scatter(128)(h, src2d, dst2d, z128)

    large = (ptr[1:] - ptr[:-1] >= 40).astype(jnp.float32).reshape(1, G)
    return _tc2(h, part2[0], part2[1], rdeg, batch.reshape(N, 1),
                large, Wl2, Wr2, b2.reshape(1, H),
                W_lin1, b_lin1.reshape(1, HD8),
                gamma.reshape(1, HD8), beta.reshape(1, HD8),
                W_lin2, b_lin2.reshape(1, C))


# speedup vs baseline: 9.4578x; 1.0470x over previous
import functools
import math

import jax
import jax.numpy as jnp
from jax import lax
from jax.experimental import pallas as pl
from jax.experimental.pallas import tpu as pltpu
from jax.experimental.pallas import tpu_sc as plsc

N = 10000
E = 320000
D = 128
H = 128
G = 64
HD8 = H // 8
C = 2

NC = 2
NS = 16
NW = NC * NS
CH = 100
EPW = E // NW
NCH = EPW // CH
RPT = N // NS

_LEAKY = 0.2
_BN_SCALE = 1.0 / math.sqrt(1.0 + 1e-5)


def _leaky(v):
    return jnp.where(v > 0, v, _LEAKY * v)


@functools.lru_cache(maxsize=None)
def _make_sc_scatter(W):
    mesh = plsc.VectorSubcoreMesh(
        core_axis_name="c", subcore_axis_name="s", num_cores=NC, num_subcores=NS
    )

    @functools.partial(
        pl.kernel,
        out_type=jax.ShapeDtypeStruct((NC, N, W), jnp.float32),
        mesh=mesh,
        scratch_types=[
            pltpu.VMEM((2, CH), jnp.int32),
            pltpu.VMEM((NCH, CH), jnp.int32),
            pltpu.VMEM((2, CH, W), jnp.float32),
            pltpu.VMEM_SHARED((N, W), jnp.float32),
            [pltpu.SemaphoreType.DMA] * 2,
            [pltpu.SemaphoreType.DMA] * 2,
            [pltpu.SemaphoreType.DMA] * 2,
        ],
        compiler_params=pltpu.CompilerParams(use_tc_tiling_on_sc=False),
    )
    def sc_pass(table_hbm, src_hbm, dst_hbm, zeros_hbm, out_hbm,
                sidx, didx, rbuf, acc_sh, isems, gsems, ssems):
        c = lax.axis_index("c")
        s = lax.axis_index("s")
        w = c * NS + s

        def idx_copy(j, p):
            pltpu.async_copy(src_hbm.at[w, j], sidx.at[p], isems[p])

        def idx_wait(j, p):
            pltpu.make_async_copy(src_hbm.at[w, j], sidx.at[p],
                                  isems[p]).wait()

        def gather(p):
            pltpu.async_copy(table_hbm.at[sidx.at[p]], rbuf.at[p], gsems[p])

        def gather_wait(p):
            pltpu.make_async_copy(table_hbm.at[sidx.at[p]], rbuf.at[p],
                                  gsems[p]).wait()

        def scatter(j, p):
            pltpu.async_copy(rbuf.at[p], acc_sh.at[didx.at[j]], ssems[p],
                             add=True)

        def scatter_wait(j, p):
            pltpu.make_async_copy(rbuf.at[p], acc_sh.at[didx.at[j]],
                                  ssems[p]).wait()

        pltpu.sync_copy(zeros_hbm.at[pl.ds(s * RPT, RPT)],
                        acc_sh.at[pl.ds(s * RPT, RPT)])
        pltpu.sync_copy(dst_hbm.at[w], didx)
        idx_copy(0, 0)
        idx_copy(1, 1)
        idx_wait(0, 0)
        gather(0)
        plsc.subcore_barrier()

        def pair_body(i, carry):
            for p in range(2):
                j = 2 * i + p
                gather_wait(p)
                scatter(j, p)

                @pl.when(j + 2 < NCH)
                def _():
                    idx_copy(j + 2, p)

                @pl.when(j >= 1)
                def _():
                    scatter_wait(j - 1, 1 - p)

                @pl.when(j + 1 < NCH)
                def _():
                    idx_wait(j + 1, 1 - p)
                    gather(1 - p)
            return carry

        lax.fori_loop(0, NCH // 2, pair_body, 0)
        scatter_wait(NCH - 1, 1)
        plsc.subcore_barrier()

        pltpu.sync_copy(acc_sh.at[pl.ds(s * RPT, RPT)],
                        out_hbm.at[c, pl.ds(s * RPT, RPT)])

    return sc_pass


_B1 = 1000
_G1 = N // _B1


def _tc1_body(x_ref, p_ref, wl_ref, wr_ref, b_ref, h_ref, rdeg_ref):
    agg = p_ref[0] + p_ref[1]
    deg = agg[:, D:D + 1]
    rdeg = 1.0 / jnp.maximum(deg, 1.0)
    mean = agg[:, :D] * rdeg
    acc = lax.dot_general(mean, wl_ref[...], (((1,), (1,)), ((), ())),
                          preferred_element_type=jnp.float32)
    acc += lax.dot_general(x_ref[...], wr_ref[...], (((1,), (1,)), ((), ())),
                           preferred_element_type=jnp.float32)
    h_ref[...] = _leaky(acc + b_ref[...])
    rdeg_ref[...] = rdeg


def _tc1(x, p, wl, wr, b):
    return pl.pallas_call(
        _tc1_body,
        grid=(_G1,),
        in_specs=[
            pl.BlockSpec((_B1, D), lambda i: (i, 0)),
            pl.BlockSpec((NC, _B1, 144), lambda i: (0, i, 0)),
            pl.BlockSpec((H, D), lambda i: (0, 0)),
            pl.BlockSpec((H, D), lambda i: (0, 0)),
            pl.BlockSpec((1, H), lambda i: (0, 0)),
        ],
        out_specs=[
            pl.BlockSpec((_B1, H), lambda i: (i, 0)),
            pl.BlockSpec((_B1, 1), lambda i: (i, 0)),
        ],
        out_shape=[
            jax.ShapeDtypeStruct((N, H), jnp.float32),
            jax.ShapeDtypeStruct((N, 1), jnp.float32),
        ],
    )(x, p, wl, wr, b)


def _tc2_body(h_ref, p_ref, rdeg_ref, batch_ref, large_ref,
              wl_ref, wr_ref, b_ref, w1_ref, b1_ref, g_ref, be_ref,
              w2_ref, b2_ref, out_ref, pool_ref, cnt_ref):
    step = pl.program_id(0)
    hb = h_ref[...]
    mean2 = (p_ref[0] + p_ref[1]) * rdeg_ref[...]
    acc = lax.dot_general(mean2, wl_ref[...], (((1,), (1,)), ((), ())),
                          preferred_element_type=jnp.float32)
    acc += lax.dot_general(hb, wr_ref[...], (((1,), (1,)), ((), ())),
                           preferred_element_type=jnp.float32)
    h2 = _leaky(acc + b_ref[...])

    iota = lax.broadcasted_iota(jnp.int32, (_B1, G), 1)
    m = (batch_ref[...] == iota).astype(jnp.float32)
    is_large = lax.dot_general(m, large_ref[...], (((1,), (1,)), ((), ())),
                               preferred_element_type=jnp.float32)
    hf = is_large * h2 + (1.0 - is_large) * hb

    ps = lax.dot_general(m, hf, (((0,), (0,)), ((), ())),
                         preferred_element_type=jnp.float32)
    cs = lax.dot_general(m, jnp.ones_like(hf), (((0,), (0,)), ((), ())),
                         preferred_element_type=jnp.float32)

    @pl.when(step == 0)
    def _():
        pool_ref[...] = jnp.zeros_like(pool_ref)
        cnt_ref[...] = jnp.zeros_like(cnt_ref)

    pool_ref[...] += ps
    cnt_ref[...] += cs

    @pl.when(step == _G1 - 1)
    def _():
        pooled = pool_ref[...] / jnp.maximum(cnt_ref[...], 1.0)
        z = lax.dot_general(pooled, w1_ref[...], (((1,), (1,)), ((), ())),
                            preferred_element_type=jnp.float32) + b1_ref[...]
        z = z * _BN_SCALE * g_ref[...] + be_ref[...]
        z = _leaky(z)
        out_ref[...] = lax.dot_general(z, w2_ref[...], (((1,), (1,)), ((), ())),
                                       preferred_element_type=jnp.float32) + b2_ref[...]


def _tc2(h, p, rdeg, batch2d, large, wl, wr, b, w1, b1, g, be, w2, b2):
    return pl.pallas_call(
        _tc2_body,
        grid=(_G1,),
        in_specs=[
            pl.BlockSpec((_B1, H), lambda i: (i, 0)),
            pl.BlockSpec((NC, _B1, H), lambda i: (0, i, 0)),
            pl.BlockSpec((_B1, 1), lambda i: (i, 0)),
            pl.BlockSpec((_B1, 1), lambda i: (i, 0)),
            pl.BlockSpec((1, G), lambda i: (0, 0)),
            pl.BlockSpec((H, H), lambda i: (0, 0)),
            pl.BlockSpec((H, H), lambda i: (0, 0)),
            pl.BlockSpec((1, H), lambda i: (0, 0)),
            pl.BlockSpec((HD8, H), lambda i: (0, 0)),
            pl.BlockSpec((1, HD8), lambda i: (0, 0)),
            pl.BlockSpec((1, HD8), lambda i: (0, 0)),
            pl.BlockSpec((1, HD8), lambda i: (0, 0)),
            pl.BlockSpec((C, HD8), lambda i: (0, 0)),
            pl.BlockSpec((1, C), lambda i: (0, 0)),
        ],
        out_specs=pl.BlockSpec((G, C), lambda i: (0, 0)),
        out_shape=jax.ShapeDtypeStruct((G, C), jnp.float32),
        scratch_shapes=[
            pltpu.VMEM((G, H), jnp.float32),
            pltpu.VMEM((G, H), jnp.float32),
        ],
    )(h, p, rdeg, batch2d, large, wl, wr, b, w1, b1, g, be, w2, b2)


def kernel(x, edge_index, batch, ptr, Wl1, Wr1, b1, Wl2, Wr2, b2,
           W_lin1, b_lin1, gamma, beta, W_lin2, b_lin2):
    src2d = edge_index[0].reshape(NW, NCH, CH)
    dst2d = edge_index[1].reshape(NW, NCH, CH)

    xpad = jnp.concatenate(
        [x, jnp.ones((N, 1), jnp.float32), jnp.zeros((N, 15), jnp.float32)],
        axis=1)
    z144 = jnp.zeros((N, 144), jnp.float32)
    z128 = jnp.zeros((N, H), jnp.float32)

    part1 = _make_sc_scatter(144)(xpad, src2d, dst2d, z144)
    h, rdeg = _tc1(x, part1, Wl1, Wr1, b1.reshape(1, H))

    part2 = _make_sc_scatter(128)(h, src2d, dst2d, z128)

    large = (ptr[1:] - ptr[:-1] >= 40).astype(jnp.float32).reshape(1, G)
    return _tc2(h, part2, rdeg, batch.reshape(N, 1),
                large, Wl2, Wr2, b2.reshape(1, H),
                W_lin1, b_lin1.reshape(1, HD8),
                gamma.reshape(1, HD8), beta.reshape(1, HD8),
                W_lin2, b_lin2.reshape(1, C))
